# baseline (speedup 1.0000x reference)
import jax
import jax.numpy as jnp
import numpy as np
from jax.experimental import pallas as pl
from jax.experimental.pallas import tpu as pltpu

_B, _S, _D, _H, _INNER, _FFN, _N = 4, 512, 1024, 8, 256, 4096, 32
_DH = _D // _H
_EPS = 1e-6

_ROWS = _B * _S
_TR = 256
_TSEQ = 64
_TA = 64

_BF = jnp.bfloat16
_F32 = jnp.float32


def _rmsnorm(x, w):
    n = jnp.sqrt(jnp.mean(x * x, axis=-1, keepdims=True))
    return w * x / (n + _EPS)


def _dot(a, b):
    return jnp.dot(a, b, preferred_element_type=_F32)


def _full_vmem():
    return pl.BlockSpec(memory_space=pltpu.VMEM)


def _pre_kernel(x_ref, wq, wk, wv, wz, w1, wc, w2, n1w, bq, bk, bv, bz, bmdi, b2,
                q_out, z_out, v_out, g_out):
    x = x_ref[...]
    xn = _rmsnorm(x, n1w[...]).astype(_BF)
    q = _dot(xn, wq[...]) + bq[...]
    k = _dot(xn, wk[...]) + bk[...]
    v = _dot(xn, wv[...]) + bv[...]
    z = _dot(xn, wz[...]) + bz[...]
    hid = jnp.tanh(_dot(z.astype(_BF), w1[...])
                   + _dot(k.astype(_BF), wc[...])
                   + bmdi[...])
    g = jax.nn.sigmoid(_dot(hid.astype(_BF), w2[...]) + b2[...])
    q_out[...] = q
    z_out[...] = z
    v_out[...] = v
    g_out[...] = g


def _seq_kernel(z_ref, g_ref, wg, bg, h_out, h_ref):
    c = pl.program_id(0)

    @pl.when(c == 0)
    def _():
        h_ref[...] = jnp.zeros_like(h_ref)

    wg_v = wg[...]
    bg_v = bg[...]
    h_prev = h_ref[...]
    for j in range(_TSEQ):
        z = z_ref[:, j, :]
        gamma = g_ref[:, j, :]
        gate = jax.nn.sigmoid(_dot(h_prev.astype(_BF), wg_v) + bg_v)
        h_new = gamma * h_prev + (1.0 - gamma) * z
        h_t = z * gate + gamma * h_prev + 0.1 * h_new
        h_out[:, j, :] = h_t
        h_prev = h_t
    h_ref[...] = h_prev


def _attn_kernel(q_ref, hp_ref, hc_ref, c_out):
    st = pl.program_id(1)
    scale = np.float32(1.0 / np.sqrt(_DH))
    q = q_ref[0]
    hp = hp_ref[0]
    hc = hc_ref[0]

    i = jax.lax.broadcasted_iota(jnp.int32, (_TA, _TA + _N), 0)
    j = jax.lax.broadcasted_iota(jnp.int32, (_TA, _TA + _N), 1)
    in_window = (j >= i) & (j < i + _N)
    zero_col = (st == 0) & (j < _N)
    neg_inf = np.float32(-np.inf)

    for h in range(_H):
        sl = slice(h * _DH, (h + 1) * _DH)
        qh = q[:, sl].astype(_BF)
        he = jnp.concatenate(
            [hp[:, sl], hc[:, sl]], axis=0).astype(_BF)
        s = jax.lax.dot_general(
            qh, he, (((1,), (1,)), ((), ())),
            preferred_element_type=_F32) * scale
        s = jnp.where(in_window, jnp.where(zero_col, 0.0, s), neg_inf)
        mx = jnp.max(s, axis=-1, keepdims=True)
        e = jnp.exp(s - mx)
        a = e / jnp.sum(e, axis=-1, keepdims=True)
        a = jnp.where(zero_col, 0.0, a)
        c_out[0, :, sl] = _dot(a.astype(_BF), he)


def _post_kernel(x_ref, h_ref, c_ref, v_ref, wo, w1f, w2f, n2w, bo, b1f, b2f,
                 out_ref):
    ctxp = _dot(c_ref[...].astype(_BF), wo[...]) + bo[...]
    h_att = h_ref[...] + ctxp + 0.1 * v_ref[...]
    x_res = x_ref[...] + h_att
    xn2 = _rmsnorm(x_res, n2w[...])
    hidf = _dot(xn2.astype(_BF), w1f[...]) + b1f[...]
    hidf = 0.5 * hidf * (1.0 + jax.lax.erf(hidf * np.float32(1.0 / np.sqrt(2.0))))
    ffn = _dot(hidf.astype(_BF), w2f[...]) + b2f[...]
    out_ref[...] = x_res + ffn


def kernel(x, mem_k, mem_v, params):
    p = params
    del mem_k, mem_v
    xr = x.reshape(_ROWS, _D)

    def b2d(name):
        return p[name][None, :]

    def wbf(name):
        return p[name].astype(_BF)

    row_spec = pl.BlockSpec((_TR, _D), lambda i: (i, 0))

    q, z, v, g = pl.pallas_call(
        _pre_kernel,
        grid=(_ROWS // _TR,),
        in_specs=[row_spec] + [_full_vmem()] * 14,
        out_specs=[row_spec] * 4,
        out_shape=[jax.ShapeDtypeStruct((_ROWS, _D), _F32)] * 4,
        compiler_params=pltpu.CompilerParams(
            dimension_semantics=("parallel",),
            vmem_limit_bytes=56 * 1024 * 1024,
        ),
        name="mmrec_pre",
    )(xr, wbf('W_q'), wbf('W_k'), wbf('W_v'), wbf('W_z'),
      wbf('W1_mdi'), wbf('Wc_mdi'), wbf('W2_mdi'),
      p['norm1_w'][None, :], b2d('b_q'), b2d('b_k'), b2d('b_v'), b2d('b_z'),
      (p['b1_mdi'] + p['bc_mdi'])[None, :], b2d('b2_mdi'))

    zb = z.reshape(_B, _S, _D)
    gb = g.reshape(_B, _S, _D)
    seq_spec = pl.BlockSpec((_B, _TSEQ, _D), lambda i: (0, i, 0))
    h_all = pl.pallas_call(
        _seq_kernel,
        grid=(_S // _TSEQ,),
        in_specs=[seq_spec, seq_spec, _full_vmem(), _full_vmem()],
        out_specs=seq_spec,
        out_shape=jax.ShapeDtypeStruct((_B, _S, _D), _F32),
        scratch_shapes=[pltpu.VMEM((_B, _D), _F32)],
        compiler_params=pltpu.CompilerParams(
            dimension_semantics=("arbitrary",),
            vmem_limit_bytes=40 * 1024 * 1024,
        ),
        name="mmrec_seq",
    )(zb, gb, wbf('W_g'), b2d('b_g'))

    qb = q.reshape(_B, _S, _D)
    ctx = pl.pallas_call(
        _attn_kernel,
        grid=(_B, _S // _TA),
        in_specs=[
            pl.BlockSpec((1, _TA, _D), lambda b, s: (b, s, 0)),
            pl.BlockSpec((1, _N, _D),
                         lambda b, s: (b, jnp.maximum(s * (_TA // _N) - 1, 0), 0)),
            pl.BlockSpec((1, _TA, _D), lambda b, s: (b, s, 0)),
        ],
        out_specs=pl.BlockSpec((1, _TA, _D), lambda b, s: (b, s, 0)),
        out_shape=jax.ShapeDtypeStruct((_B, _S, _D), _F32),
        compiler_params=pltpu.CompilerParams(
            dimension_semantics=("parallel", "arbitrary"),
            vmem_limit_bytes=40 * 1024 * 1024,
        ),
        name="mmrec_attn",
    )(qb, h_all, h_all)

    out = pl.pallas_call(
        _post_kernel,
        grid=(_ROWS // _TR,),
        in_specs=[row_spec] * 4 + [_full_vmem()] * 7,
        out_specs=row_spec,
        out_shape=jax.ShapeDtypeStruct((_ROWS, _D), _F32),
        compiler_params=pltpu.CompilerParams(
            dimension_semantics=("parallel",),
            vmem_limit_bytes=56 * 1024 * 1024,
        ),
        name="mmrec_post",
    )(xr, h_all.reshape(_ROWS, _D), ctx.reshape(_ROWS, _D), v,
      wbf('Wo_attn'), wbf('W_ffn1'), wbf('W_ffn2'), p['norm2_w'][None, :],
      b2d('bo_attn'), b2d('b_ffn1'), b2d('b_ffn2'))

    return out.reshape(_B, _S, _D)


# speedup vs baseline: 1.3892x; 1.3892x over previous
import jax
import jax.numpy as jnp
import numpy as np
from jax.experimental import pallas as pl
from jax.experimental.pallas import tpu as pltpu

_B, _S, _D, _H, _INNER, _FFN, _N = 4, 512, 1024, 8, 256, 4096, 32
_DH = _D // _H
_EPS = 1e-6

_ROWS = _B * _S
_TR = 256
_TSEQ = 32
_TA = 64

_BF = jnp.bfloat16
_F32 = jnp.float32


def _rmsnorm(x, w):
    n = jnp.sqrt(jnp.mean(x * x, axis=-1, keepdims=True))
    return w * x / (n + _EPS)


def _dot(a, b):
    return jnp.dot(a, b, preferred_element_type=_F32)


def _full_vmem():
    return pl.BlockSpec(memory_space=pltpu.VMEM)


def _pre_kernel(x_ref, wq, wk, wv, wz, w1, wc, w2, n1w, bq, bk, bv, bz, bmdi, b2,
                q_out, z_out, v_out, g_out):
    x = x_ref[...]
    xn = _rmsnorm(x, n1w[...]).astype(_BF)
    q = _dot(xn, wq[...]) + bq[...]
    k = _dot(xn, wk[...]) + bk[...]
    v = _dot(xn, wv[...]) + bv[...]
    z = _dot(xn, wz[...]) + bz[...]
    hid = jnp.tanh(_dot(z.astype(_BF), w1[...])
                   + _dot(k.astype(_BF), wc[...])
                   + bmdi[...])
    g = jax.nn.sigmoid(_dot(hid.astype(_BF), w2[...]) + b2[...])
    q_out[...] = q
    z_out[...] = z
    v_out[...] = v
    g_out[...] = g


_F8 = jnp.float8_e4m3fn
_WG_SCALE = 64.0


def _seq_kernel(z_ref, g_ref, wg, bg, h_out, h_ref):
    c = pl.program_id(0)

    @pl.when(c == 0)
    def _():
        h_ref[...] = jnp.zeros_like(h_ref)

    wg_a = wg[:, : _D // 2]
    wg_b = wg[:, _D // 2:]
    bg_v = bg[...]
    inv = np.float32(1.0 / _WG_SCALE)
    h_prev = h_ref[...]
    for j in range(_TSEQ):
        z = z_ref[:, j, :]
        gamma = g_ref[:, j, :]
        hq = h_prev.astype(_F8)
        la = _dot(hq, wg_a) * inv
        lb = _dot(hq, wg_b) * inv
        gate = jax.nn.sigmoid(jnp.concatenate([la, lb], axis=-1) + bg_v)
        h_new = gamma * h_prev + (1.0 - gamma) * z
        h_t = z * gate + gamma * h_prev + 0.1 * h_new
        h_out[:, j, :] = h_t
        h_prev = h_t
    h_ref[...] = h_prev


def _attn_kernel(q_ref, hp_ref, hc_ref, c_out):
    st = pl.program_id(1)
    scale = np.float32(1.0 / np.sqrt(_DH))
    q = q_ref[0]
    hp = hp_ref[0]
    hc = hc_ref[0]

    i = jax.lax.broadcasted_iota(jnp.int32, (_TA, _TA + _N), 0)
    j = jax.lax.broadcasted_iota(jnp.int32, (_TA, _TA + _N), 1)
    in_window = (j >= i) & (j < i + _N)
    zero_col = (st == 0) & (j < _N)
    neg_inf = np.float32(-np.inf)

    for h in range(_H):
        sl = slice(h * _DH, (h + 1) * _DH)
        qh = q[:, sl].astype(_BF)
        he = jnp.concatenate(
            [hp[:, sl], hc[:, sl]], axis=0).astype(_BF)
        s = jax.lax.dot_general(
            qh, he, (((1,), (1,)), ((), ())),
            preferred_element_type=_F32) * scale
        s = jnp.where(in_window, jnp.where(zero_col, 0.0, s), neg_inf)
        mx = jnp.max(s, axis=-1, keepdims=True)
        e = jnp.exp(s - mx)
        a = e / jnp.sum(e, axis=-1, keepdims=True)
        a = jnp.where(zero_col, 0.0, a)
        c_out[0, :, sl] = _dot(a.astype(_BF), he)


def _post_kernel(x_ref, h_ref, c_ref, v_ref, wo, w1f, w2f, n2w, bo, b1f, b2f,
                 out_ref):
    ctxp = _dot(c_ref[...].astype(_BF), wo[...]) + bo[...]
    h_att = h_ref[...] + ctxp + 0.1 * v_ref[...]
    x_res = x_ref[...] + h_att
    xn2 = _rmsnorm(x_res, n2w[...])
    hidf = _dot(xn2.astype(_BF), w1f[...]) + b1f[...]
    hidf = 0.5 * hidf * (1.0 + jax.lax.erf(hidf * np.float32(1.0 / np.sqrt(2.0))))
    ffn = _dot(hidf.astype(_BF), w2f[...]) + b2f[...]
    out_ref[...] = x_res + ffn


def kernel(x, mem_k, mem_v, params):
    p = params
    del mem_k, mem_v
    xr = x.reshape(_ROWS, _D)

    def b2d(name):
        return p[name][None, :]

    def wbf(name):
        return p[name].astype(_BF)

    row_spec = pl.BlockSpec((_TR, _D), lambda i: (i, 0))

    q, z, v, g = pl.pallas_call(
        _pre_kernel,
        grid=(_ROWS // _TR,),
        in_specs=[row_spec] + [_full_vmem()] * 14,
        out_specs=[row_spec] * 4,
        out_shape=[jax.ShapeDtypeStruct((_ROWS, _D), _F32)] * 4,
        compiler_params=pltpu.CompilerParams(
            dimension_semantics=("parallel",),
            vmem_limit_bytes=56 * 1024 * 1024,
        ),
        name="mmrec_pre",
    )(xr, wbf('W_q'), wbf('W_k'), wbf('W_v'), wbf('W_z'),
      wbf('W1_mdi'), wbf('Wc_mdi'), wbf('W2_mdi'),
      p['norm1_w'][None, :], b2d('b_q'), b2d('b_k'), b2d('b_v'), b2d('b_z'),
      (p['b1_mdi'] + p['bc_mdi'])[None, :], b2d('b2_mdi'))

    zb = z.reshape(_B, _S, _D)
    gb = g.reshape(_B, _S, _D)
    seq_spec = pl.BlockSpec((_B, _TSEQ, _D), lambda i: (0, i, 0))
    h_all = pl.pallas_call(
        _seq_kernel,
        grid=(_S // _TSEQ,),
        in_specs=[seq_spec, seq_spec, _full_vmem(), _full_vmem()],
        out_specs=seq_spec,
        out_shape=jax.ShapeDtypeStruct((_B, _S, _D), _F32),
        scratch_shapes=[pltpu.VMEM((_B, _D), _F32)],
        compiler_params=pltpu.CompilerParams(
            dimension_semantics=("arbitrary",),
            vmem_limit_bytes=40 * 1024 * 1024,
        ),
        name="mmrec_seq",
    )(zb, gb, (p['W_g'] * _WG_SCALE).astype(_F8), b2d('b_g'))

    qb = q.reshape(_B, _S, _D)
    ctx = pl.pallas_call(
        _attn_kernel,
        grid=(_B, _S // _TA),
        in_specs=[
            pl.BlockSpec((1, _TA, _D), lambda b, s: (b, s, 0)),
            pl.BlockSpec((1, _N, _D),
                         lambda b, s: (b, jnp.maximum(s * (_TA // _N) - 1, 0), 0)),
            pl.BlockSpec((1, _TA, _D), lambda b, s: (b, s, 0)),
        ],
        out_specs=pl.BlockSpec((1, _TA, _D), lambda b, s: (b, s, 0)),
        out_shape=jax.ShapeDtypeStruct((_B, _S, _D), _F32),
        compiler_params=pltpu.CompilerParams(
            dimension_semantics=("parallel", "arbitrary"),
            vmem_limit_bytes=40 * 1024 * 1024,
        ),
        name="mmrec_attn",
    )(qb, h_all, h_all)

    out = pl.pallas_call(
        _post_kernel,
        grid=(_ROWS // _TR,),
        in_specs=[row_spec] * 4 + [_full_vmem()] * 7,
        out_specs=row_spec,
        out_shape=jax.ShapeDtypeStruct((_ROWS, _D), _F32),
        compiler_params=pltpu.CompilerParams(
            dimension_semantics=("parallel",),
            vmem_limit_bytes=56 * 1024 * 1024,
        ),
        name="mmrec_post",
    )(xr, h_all.reshape(_ROWS, _D), ctx.reshape(_ROWS, _D), v,
      wbf('Wo_attn'), wbf('W_ffn1'), wbf('W_ffn2'), p['norm2_w'][None, :],
      b2d('bo_attn'), b2d('b_ffn1'), b2d('b_ffn2'))

    return out.reshape(_B, _S, _D)


# speedup vs baseline: 1.4911x; 1.0733x over previous
import jax
import jax.numpy as jnp
import numpy as np
from jax.experimental import pallas as pl
from jax.experimental.pallas import tpu as pltpu

_B, _S, _D, _H, _INNER, _FFN, _N = 4, 512, 1024, 8, 256, 4096, 32
_DH = _D // _H
_EPS = 1e-6

_ROWS = _B * _S
_TR = 256
_TSEQ = 32
_TA = 64

_BF = jnp.bfloat16
_F32 = jnp.float32


def _rmsnorm(x, w):
    n = jnp.sqrt(jnp.mean(x * x, axis=-1, keepdims=True))
    return w * x / (n + _EPS)


def _dot(a, b):
    return jnp.dot(a, b, preferred_element_type=_F32)


def _full_vmem():
    return pl.BlockSpec(memory_space=pltpu.VMEM)


def _pre_kernel(x_ref, wq, wk, wv, wz, w1, wc, w2, n1w, bq, bk, bv, bz, bmdi, b2,
                q_out, z_out, v_out, g_out):
    x = x_ref[...]
    xn = _rmsnorm(x, n1w[...]).astype(_BF)
    q = _dot(xn, wq[...]) + bq[...]
    k = _dot(xn, wk[...]) + bk[...]
    v = _dot(xn, wv[...]) + bv[...]
    z = _dot(xn, wz[...]) + bz[...]
    hid = jnp.tanh(_dot(z.astype(_BF), w1[...])
                   + _dot(k.astype(_BF), wc[...])
                   + bmdi[...])
    g = jax.nn.sigmoid(_dot(hid.astype(_BF), w2[...]) + b2[...])
    q_out[...] = q
    z_out[...] = z
    v_out[...] = v
    g_out[...] = g


_F8 = jnp.float8_e4m3fn
_WG_SCALE = 64.0


def _seq_kernel(z_ref, g_ref, wg, bg, h_out, h_ref):
    c = pl.program_id(0)

    @pl.when(c == 0)
    def _():
        h_ref[...] = jnp.zeros_like(h_ref)

    hw = _D // 2
    wg_a = wg[:, :hw]
    wg_b = wg[:, hw:]
    bg_a = bg[:, :hw]
    bg_b = bg[:, hw:]
    inv = np.float32(1.0 / _WG_SCALE)
    h_prev = h_ref[...]
    for j in range(_TSEQ):
        hq = h_prev.astype(_F8)
        la = _dot(hq, wg_a)
        lb = _dot(hq, wg_b)
        halves = []
        for lo, logit, bgh in ((0, la, bg_a), (hw, lb, bg_b)):
            z = z_ref[:, j, lo:lo + hw]
            gamma = g_ref[:, j, lo:lo + hw]
            hp = h_prev[:, lo:lo + hw]
            gate = jax.nn.sigmoid(logit * inv + bgh)
            h_new = gamma * hp + (1.0 - gamma) * z
            h_half = z * gate + gamma * hp + 0.1 * h_new
            h_out[:, j, lo:lo + hw] = h_half
            halves.append(h_half)
        h_prev = jnp.concatenate(halves, axis=-1)
    h_ref[...] = h_prev


def _attn_kernel(q_ref, hp_ref, hc_ref, c_out):
    st = pl.program_id(1)
    scale = np.float32(1.0 / np.sqrt(_DH))
    q = q_ref[0]
    hp = hp_ref[0]
    hc = hc_ref[0]

    i = jax.lax.broadcasted_iota(jnp.int32, (_TA, _TA + _N), 0)
    j = jax.lax.broadcasted_iota(jnp.int32, (_TA, _TA + _N), 1)
    in_window = (j >= i) & (j < i + _N)
    zero_col = (st == 0) & (j < _N)
    neg_inf = np.float32(-np.inf)

    hes, ss = [], []
    for h in range(_H):
        sl = slice(h * _DH, (h + 1) * _DH)
        qh = q[:, sl].astype(_BF)
        he = jnp.concatenate(
            [hp[:, sl], hc[:, sl]], axis=0).astype(_BF)
        hes.append(he)
        ss.append(jax.lax.dot_general(
            qh, he, (((1,), (1,)), ((), ())),
            preferred_element_type=_F32) * scale)
    aas = []
    for h in range(_H):
        s = jnp.where(in_window, jnp.where(zero_col, 0.0, ss[h]), neg_inf)
        mx = jnp.max(s, axis=-1, keepdims=True)
        e = jnp.exp(s - mx)
        a = e / jnp.sum(e, axis=-1, keepdims=True)
        aas.append(jnp.where(zero_col, 0.0, a).astype(_BF))
    for h in range(_H):
        sl = slice(h * _DH, (h + 1) * _DH)
        c_out[0, :, sl] = _dot(aas[h], hes[h])


def _post_kernel(x_ref, h_ref, c_ref, v_ref, wo, w1f, w2f, n2w, bo, b1f, b2f,
                 out_ref):
    ctxp = _dot(c_ref[...].astype(_BF), wo[...]) + bo[...]
    h_att = h_ref[...] + ctxp + 0.1 * v_ref[...]
    x_res = x_ref[...] + h_att
    xn2 = _rmsnorm(x_res, n2w[...])
    hidf = _dot(xn2.astype(_BF), w1f[...]) + b1f[...]
    hidf = 0.5 * hidf * (1.0 + jax.lax.erf(hidf * np.float32(1.0 / np.sqrt(2.0))))
    ffn = _dot(hidf.astype(_BF), w2f[...]) + b2f[...]
    out_ref[...] = x_res + ffn


def kernel(x, mem_k, mem_v, params):
    p = params
    del mem_k, mem_v
    xr = x.reshape(_ROWS, _D)

    def b2d(name):
        return p[name][None, :]

    def wbf(name):
        return p[name].astype(_BF)

    def wf8(name):
        return (p[name] * _WG_SCALE).astype(_F8)

    row_spec = pl.BlockSpec((_TR, _D), lambda i: (i, 0))

    q, z, v, g = pl.pallas_call(
        _pre_kernel,
        grid=(_ROWS // _TR,),
        in_specs=[row_spec] + [_full_vmem()] * 14,
        out_specs=[row_spec] * 4,
        out_shape=[jax.ShapeDtypeStruct((_ROWS, _D), _F32)] * 4,
        compiler_params=pltpu.CompilerParams(
            dimension_semantics=("parallel",),
            vmem_limit_bytes=56 * 1024 * 1024,
        ),
        name="mmrec_pre",
    )(xr, wbf('W_q'), wbf('W_k'), wbf('W_v'), wbf('W_z'),
      wbf('W1_mdi'), wbf('Wc_mdi'), wbf('W2_mdi'),
      p['norm1_w'][None, :], b2d('b_q'), b2d('b_k'), b2d('b_v'), b2d('b_z'),
      (p['b1_mdi'] + p['bc_mdi'])[None, :], b2d('b2_mdi'))

    zb = z.reshape(_B, _S, _D)
    gb = g.reshape(_B, _S, _D)
    seq_spec = pl.BlockSpec((_B, _TSEQ, _D), lambda i: (0, i, 0))
    h_all = pl.pallas_call(
        _seq_kernel,
        grid=(_S // _TSEQ,),
        in_specs=[seq_spec, seq_spec, _full_vmem(), _full_vmem()],
        out_specs=seq_spec,
        out_shape=jax.ShapeDtypeStruct((_B, _S, _D), _F32),
        scratch_shapes=[pltpu.VMEM((_B, _D), _F32)],
        compiler_params=pltpu.CompilerParams(
            dimension_semantics=("arbitrary",),
            vmem_limit_bytes=40 * 1024 * 1024,
        ),
        name="mmrec_seq",
    )(zb, gb, wf8('W_g'), b2d('b_g'))

    qb = q.reshape(_B, _S, _D)
    ctx = pl.pallas_call(
        _attn_kernel,
        grid=(_B, _S // _TA),
        in_specs=[
            pl.BlockSpec((1, _TA, _D), lambda b, s: (b, s, 0)),
            pl.BlockSpec((1, _N, _D),
                         lambda b, s: (b, jnp.maximum(s * (_TA // _N) - 1, 0), 0)),
            pl.BlockSpec((1, _TA, _D), lambda b, s: (b, s, 0)),
        ],
        out_specs=pl.BlockSpec((1, _TA, _D), lambda b, s: (b, s, 0)),
        out_shape=jax.ShapeDtypeStruct((_B, _S, _D), _F32),
        compiler_params=pltpu.CompilerParams(
            dimension_semantics=("parallel", "arbitrary"),
            vmem_limit_bytes=40 * 1024 * 1024,
        ),
        name="mmrec_attn",
    )(qb, h_all, h_all)

    out = pl.pallas_call(
        _post_kernel,
        grid=(_ROWS // _TR,),
        in_specs=[row_spec] * 4 + [_full_vmem()] * 7,
        out_specs=row_spec,
        out_shape=jax.ShapeDtypeStruct((_ROWS, _D), _F32),
        compiler_params=pltpu.CompilerParams(
            dimension_semantics=("parallel",),
            vmem_limit_bytes=56 * 1024 * 1024,
        ),
        name="mmrec_post",
    )(xr, h_all.reshape(_ROWS, _D), ctx.reshape(_ROWS, _D), v,
      wbf('Wo_attn'), wbf('W_ffn1'), wbf('W_ffn2'), p['norm2_w'][None, :],
      b2d('bo_attn'), b2d('b_ffn1'), b2d('b_ffn2'))

    return out.reshape(_B, _S, _D)
